# baseline (speedup 1.0000x reference)
import jax
import jax.numpy as jnp
from jax import lax
from jax.experimental import pallas as pl
from jax.experimental.pallas import tpu as pltpu

IN_LEN = 513
KW = 32
C_OUT = 32
T_POOL = 120
J = 35
UI = 24
NUO = 5
SLAB = 4 * UI
NCOL = 128 * UI
PGRP = C_OUT * UI
BB = 1024


MSUB = 128
NCHK = 3


def _fwd_kernel(x_ref, wsel_ref, aux_ref, fcb_ref, out_ref):
    def body(s, _):
        r0 = s * MSUB
        accs = [jnp.zeros((MSUB, 256), jnp.float32) for _ in range(NCHK)]
        for uo in range(NUO):
            xs = x_ref[pl.ds(r0, MSUB), pl.ds(SLAB * uo, 128)]
            for nc in range(NCHK):
                col = nc * 256
                d0 = jnp.dot(xs, wsel_ref[:, col:col + 256],
                             preferred_element_type=jnp.float32)
                d1 = jnp.dot(xs, wsel_ref[:, PGRP + col:PGRP + col + 256],
                             preferred_element_type=jnp.float32)
                d2 = jnp.dot(xs, wsel_ref[:, 2 * PGRP + col:2 * PGRP + col + 256],
                             preferred_element_type=jnp.float32)
                d3 = jnp.dot(xs, wsel_ref[:, 3 * PGRP + col:3 * PGRP + col + 256],
                             preferred_element_type=jnp.float32)
                m = jnp.maximum(jnp.maximum(d0, d1), jnp.maximum(d2, d3))
                act = jnp.maximum(m + aux_ref[NUO:NUO + 1, col:col + 256], 0.0)
                accs[nc] = accs[nc] + act * aux_ref[uo:uo + 1, col:col + 256]
        logit = (jnp.sum(accs[0], axis=1, keepdims=True)
                 + jnp.sum(accs[1], axis=1, keepdims=True)
                 + jnp.sum(accs[2], axis=1, keepdims=True)) + fcb_ref[0, 0]
        out_ref[pl.ds(r0, MSUB), :] = 1.0 / (1.0 + jnp.exp(-logit))
        return 0

    lax.fori_loop(0, BB // MSUB, body, 0)


def kernel(x_batch, wkT, fcwT, fcb):
    B = x_batch.shape[0]
    x = x_batch.astype(jnp.float32)

    r = lax.broadcasted_iota(jnp.int32, (128, UI, J), 0)
    ui = lax.broadcasted_iota(jnp.int32, (128, UI, J), 1)
    j = lax.broadcasted_iota(jnp.int32, (128, UI, J), 2)
    sel = (r == 4 * ui + j).astype(jnp.float32)
    wsel = jnp.einsum('ruj,cj->rcu', sel, wkT[:, :J])
    wsel = wsel.reshape(128, NCOL)

    fcv = fcwT[:, :T_POOL].reshape(C_OUT, NUO, UI)
    fcv = fcv.transpose(1, 0, 2).reshape(NUO, PGRP)
    bcol = jnp.broadcast_to(wkT[:C_OUT, J:J + 1], (C_OUT, UI)).reshape(1, PGRP)
    aux = jnp.concatenate(
        [fcv, bcol, jnp.zeros((2, PGRP), jnp.float32)], axis=0)

    nb = pl.cdiv(B, BB)
    if B % BB:
        x = jnp.pad(x, ((0, nb * BB - B), (0, 0)))

    out = pl.pallas_call(
        _fwd_kernel,
        out_shape=jax.ShapeDtypeStruct((nb * BB, 1), jnp.float32),
        grid=(nb,),
        in_specs=[
            pl.BlockSpec((BB, IN_LEN), lambda i: (i, 0)),
            pl.BlockSpec((128, NCOL), lambda i: (0, 0)),
            pl.BlockSpec((8, PGRP), lambda i: (0, 0)),
            pl.BlockSpec(memory_space=pltpu.MemorySpace.SMEM),
        ],
        out_specs=pl.BlockSpec((BB, 1), lambda i: (i, 0)),
        compiler_params=pltpu.CompilerParams(
            dimension_semantics=("arbitrary",),
            skip_device_barrier=True,
            vmem_limit_bytes=64 * 1024 * 1024),
    )(x, wsel, aux, fcb.reshape(1, 1))
    return out[:B]


# speedup vs baseline: 1.0862x; 1.0862x over previous
import jax
import jax.numpy as jnp
from jax import lax
from jax.experimental import pallas as pl
from jax.experimental.pallas import tpu as pltpu

IN_LEN = 513
KW = 32
C_OUT = 32
T_POOL = 120
J = 35
UI = 24
NUO = 5
SLAB = 4 * UI
NCOL = 128 * UI
PGRP = C_OUT * UI
BB = 1024


MSUB = 256
NCHK = 3


def _fwd_kernel(x_ref, wsel_ref, aux_ref, fcb_ref, out_ref):
    def body(s, _):
        r0 = s * MSUB
        accs = [jnp.zeros((MSUB, 256), jnp.float32) for _ in range(NCHK)]
        for uo in range(NUO):
            xs = x_ref[pl.ds(r0, MSUB), pl.ds(SLAB * uo, 128)]
            for nc in range(NCHK):
                col = nc * 256
                d0 = jnp.dot(xs, wsel_ref[:, col:col + 256],
                             preferred_element_type=jnp.float32)
                d1 = jnp.dot(xs, wsel_ref[:, PGRP + col:PGRP + col + 256],
                             preferred_element_type=jnp.float32)
                d2 = jnp.dot(xs, wsel_ref[:, 2 * PGRP + col:2 * PGRP + col + 256],
                             preferred_element_type=jnp.float32)
                d3 = jnp.dot(xs, wsel_ref[:, 3 * PGRP + col:3 * PGRP + col + 256],
                             preferred_element_type=jnp.float32)
                m = jnp.maximum(jnp.maximum(d0, d1), jnp.maximum(d2, d3))
                act = jnp.maximum(m + aux_ref[NUO:NUO + 1, col:col + 256], 0.0)
                accs[nc] = accs[nc] + act * aux_ref[uo:uo + 1, col:col + 256]
        logit = (jnp.sum(accs[0], axis=1, keepdims=True)
                 + jnp.sum(accs[1], axis=1, keepdims=True)
                 + jnp.sum(accs[2], axis=1, keepdims=True)) + fcb_ref[0, 0]
        out_ref[pl.ds(r0, MSUB), :] = 1.0 / (1.0 + jnp.exp(-logit))
        return 0

    lax.fori_loop(0, BB // MSUB, body, 0)


def kernel(x_batch, wkT, fcwT, fcb):
    B = x_batch.shape[0]
    x = x_batch.astype(jnp.float32)

    r = lax.broadcasted_iota(jnp.int32, (128, UI, J), 0)
    ui = lax.broadcasted_iota(jnp.int32, (128, UI, J), 1)
    j = lax.broadcasted_iota(jnp.int32, (128, UI, J), 2)
    sel = (r == 4 * ui + j).astype(jnp.float32)
    wsel = jnp.einsum('ruj,cj->rcu', sel, wkT[:, :J])
    wsel = wsel.reshape(128, NCOL)

    fcv = fcwT[:, :T_POOL].reshape(C_OUT, NUO, UI)
    fcv = fcv.transpose(1, 0, 2).reshape(NUO, PGRP)
    bcol = jnp.broadcast_to(wkT[:C_OUT, J:J + 1], (C_OUT, UI)).reshape(1, PGRP)
    aux = jnp.concatenate(
        [fcv, bcol, jnp.zeros((2, PGRP), jnp.float32)], axis=0)

    nb = pl.cdiv(B, BB)
    if B % BB:
        x = jnp.pad(x, ((0, nb * BB - B), (0, 0)))

    out = pl.pallas_call(
        _fwd_kernel,
        out_shape=jax.ShapeDtypeStruct((nb * BB, 1), jnp.float32),
        grid=(nb,),
        in_specs=[
            pl.BlockSpec((BB, IN_LEN), lambda i: (i, 0)),
            pl.BlockSpec((128, NCOL), lambda i: (0, 0)),
            pl.BlockSpec((8, PGRP), lambda i: (0, 0)),
            pl.BlockSpec(memory_space=pltpu.MemorySpace.SMEM),
        ],
        out_specs=pl.BlockSpec((BB, 1), lambda i: (i, 0)),
        compiler_params=pltpu.CompilerParams(
            dimension_semantics=("arbitrary",),
            skip_device_barrier=True,
            vmem_limit_bytes=64 * 1024 * 1024),
    )(x, wsel, aux, fcb.reshape(1, 1))
    return out[:B]


# speedup vs baseline: 1.1392x; 1.0488x over previous
import jax
import jax.numpy as jnp
from jax import lax
from jax.experimental import pallas as pl
from jax.experimental.pallas import tpu as pltpu

IN_LEN = 513
KW = 32
C_OUT = 32
T_POOL = 120
J = 35
UI = 24
NUO = 5
SLAB = 4 * UI
NCOL = 128 * UI
PGRP = C_OUT * UI
BB = 1024


MSUB = 512
NCHK = 3


def _fwd_kernel(x_ref, wsel_ref, aux_ref, fcb_ref, out_ref):
    def body(s, _):
        r0 = s * MSUB
        accs = [jnp.zeros((MSUB, 256), jnp.float32) for _ in range(NCHK)]
        for uo in range(NUO):
            xs = x_ref[pl.ds(r0, MSUB), pl.ds(SLAB * uo, 128)]
            for nc in range(NCHK):
                col = nc * 256
                d0 = jnp.dot(xs, wsel_ref[:, col:col + 256],
                             preferred_element_type=jnp.float32)
                d1 = jnp.dot(xs, wsel_ref[:, PGRP + col:PGRP + col + 256],
                             preferred_element_type=jnp.float32)
                d2 = jnp.dot(xs, wsel_ref[:, 2 * PGRP + col:2 * PGRP + col + 256],
                             preferred_element_type=jnp.float32)
                d3 = jnp.dot(xs, wsel_ref[:, 3 * PGRP + col:3 * PGRP + col + 256],
                             preferred_element_type=jnp.float32)
                m = jnp.maximum(jnp.maximum(d0, d1), jnp.maximum(d2, d3))
                act = jnp.maximum(m + aux_ref[NUO:NUO + 1, col:col + 256], 0.0)
                accs[nc] = accs[nc] + act * aux_ref[uo:uo + 1, col:col + 256]
        logit = (jnp.sum(accs[0], axis=1, keepdims=True)
                 + jnp.sum(accs[1], axis=1, keepdims=True)
                 + jnp.sum(accs[2], axis=1, keepdims=True)) + fcb_ref[0, 0]
        out_ref[pl.ds(r0, MSUB), :] = 1.0 / (1.0 + jnp.exp(-logit))
        return 0

    lax.fori_loop(0, BB // MSUB, body, 0)


def kernel(x_batch, wkT, fcwT, fcb):
    B = x_batch.shape[0]
    x = x_batch.astype(jnp.float32)

    r = lax.broadcasted_iota(jnp.int32, (128, UI, J), 0)
    ui = lax.broadcasted_iota(jnp.int32, (128, UI, J), 1)
    j = lax.broadcasted_iota(jnp.int32, (128, UI, J), 2)
    sel = (r == 4 * ui + j).astype(jnp.float32)
    wsel = jnp.einsum('ruj,cj->rcu', sel, wkT[:, :J])
    wsel = wsel.reshape(128, NCOL)

    fcv = fcwT[:, :T_POOL].reshape(C_OUT, NUO, UI)
    fcv = fcv.transpose(1, 0, 2).reshape(NUO, PGRP)
    bcol = jnp.broadcast_to(wkT[:C_OUT, J:J + 1], (C_OUT, UI)).reshape(1, PGRP)
    aux = jnp.concatenate(
        [fcv, bcol, jnp.zeros((2, PGRP), jnp.float32)], axis=0)

    nb = pl.cdiv(B, BB)
    if B % BB:
        x = jnp.pad(x, ((0, nb * BB - B), (0, 0)))

    out = pl.pallas_call(
        _fwd_kernel,
        out_shape=jax.ShapeDtypeStruct((nb * BB, 1), jnp.float32),
        grid=(nb,),
        in_specs=[
            pl.BlockSpec((BB, IN_LEN), lambda i: (i, 0)),
            pl.BlockSpec((128, NCOL), lambda i: (0, 0)),
            pl.BlockSpec((8, PGRP), lambda i: (0, 0)),
            pl.BlockSpec(memory_space=pltpu.MemorySpace.SMEM),
        ],
        out_specs=pl.BlockSpec((BB, 1), lambda i: (i, 0)),
        compiler_params=pltpu.CompilerParams(
            dimension_semantics=("arbitrary",),
            skip_device_barrier=True,
            vmem_limit_bytes=64 * 1024 * 1024),
    )(x, wsel, aux, fcb.reshape(1, 1))
    return out[:B]


# speedup vs baseline: 1.1555x; 1.0143x over previous
import jax
import jax.numpy as jnp
from jax import lax
from jax.experimental import pallas as pl
from jax.experimental.pallas import tpu as pltpu

IN_LEN = 513
KW = 32
C_OUT = 32
T_POOL = 120
J = 35
UI = 24
NUO = 5
SLAB = 4 * UI
NCOL = 128 * UI
PGRP = C_OUT * UI
BB = 1024


MSUB = 1024
NCHK = 3


def _fwd_kernel(x_ref, wsel_ref, aux_ref, fcb_ref, out_ref):
    def body(s, _):
        r0 = s * MSUB
        accs = [jnp.zeros((MSUB, 256), jnp.float32) for _ in range(NCHK)]
        for uo in range(NUO):
            xs = x_ref[pl.ds(r0, MSUB), pl.ds(SLAB * uo, 128)]
            for nc in range(NCHK):
                col = nc * 256
                d0 = jnp.dot(xs, wsel_ref[:, col:col + 256],
                             preferred_element_type=jnp.float32)
                d1 = jnp.dot(xs, wsel_ref[:, PGRP + col:PGRP + col + 256],
                             preferred_element_type=jnp.float32)
                d2 = jnp.dot(xs, wsel_ref[:, 2 * PGRP + col:2 * PGRP + col + 256],
                             preferred_element_type=jnp.float32)
                d3 = jnp.dot(xs, wsel_ref[:, 3 * PGRP + col:3 * PGRP + col + 256],
                             preferred_element_type=jnp.float32)
                m = jnp.maximum(jnp.maximum(d0, d1), jnp.maximum(d2, d3))
                act = jnp.maximum(m + aux_ref[NUO:NUO + 1, col:col + 256], 0.0)
                accs[nc] = accs[nc] + act * aux_ref[uo:uo + 1, col:col + 256]
        logit = (jnp.sum(accs[0], axis=1, keepdims=True)
                 + jnp.sum(accs[1], axis=1, keepdims=True)
                 + jnp.sum(accs[2], axis=1, keepdims=True)) + fcb_ref[0, 0]
        out_ref[pl.ds(r0, MSUB), :] = 1.0 / (1.0 + jnp.exp(-logit))
        return 0

    lax.fori_loop(0, BB // MSUB, body, 0)


def kernel(x_batch, wkT, fcwT, fcb):
    B = x_batch.shape[0]
    x = x_batch.astype(jnp.float32)

    r = lax.broadcasted_iota(jnp.int32, (128, UI, J), 0)
    ui = lax.broadcasted_iota(jnp.int32, (128, UI, J), 1)
    j = lax.broadcasted_iota(jnp.int32, (128, UI, J), 2)
    sel = (r == 4 * ui + j).astype(jnp.float32)
    wsel = jnp.einsum('ruj,cj->rcu', sel, wkT[:, :J])
    wsel = wsel.reshape(128, NCOL)

    fcv = fcwT[:, :T_POOL].reshape(C_OUT, NUO, UI)
    fcv = fcv.transpose(1, 0, 2).reshape(NUO, PGRP)
    bcol = jnp.broadcast_to(wkT[:C_OUT, J:J + 1], (C_OUT, UI)).reshape(1, PGRP)
    aux = jnp.concatenate(
        [fcv, bcol, jnp.zeros((2, PGRP), jnp.float32)], axis=0)

    nb = pl.cdiv(B, BB)
    if B % BB:
        x = jnp.pad(x, ((0, nb * BB - B), (0, 0)))

    out = pl.pallas_call(
        _fwd_kernel,
        out_shape=jax.ShapeDtypeStruct((nb * BB, 1), jnp.float32),
        grid=(nb,),
        in_specs=[
            pl.BlockSpec((BB, IN_LEN), lambda i: (i, 0)),
            pl.BlockSpec((128, NCOL), lambda i: (0, 0)),
            pl.BlockSpec((8, PGRP), lambda i: (0, 0)),
            pl.BlockSpec(memory_space=pltpu.MemorySpace.SMEM),
        ],
        out_specs=pl.BlockSpec((BB, 1), lambda i: (i, 0)),
        compiler_params=pltpu.CompilerParams(
            dimension_semantics=("arbitrary",),
            skip_device_barrier=True,
            vmem_limit_bytes=64 * 1024 * 1024),
    )(x, wsel, aux, fcb.reshape(1, 1))
    return out[:B]
